# baseline (speedup 1.0000x reference)
import jax
import jax.numpy as jnp
from jax import lax
from jax.experimental import pallas as pl
from jax.experimental.pallas import tpu as pltpu

N = 8192
F = 2048
TOPK = 64
NSIM = 16
EPS = 1e-12
ROWBLK = 1024
NBLK = N // ROWBLK
NEG = float("-inf")


def _fused_body(x_ref, xg_ref, scale_ref, means_s, topn_s, a_s, tidx_s, sem):
    i = pl.program_id(0)

    @pl.when(i < NBLK)
    def _stats():
        xb = x_ref[...]
        s = jnp.sum(xb, axis=1, keepdims=True)
        m2 = jnp.reshape(s * (1.0 / F), (ROWBLK // 128, 128))
        means_s[pl.ds(i * (ROWBLK // 128), ROWBLK // 128), :] = m2

    @pl.when(i == NBLK)
    def _select_gather():
        m = means_s[...]
        rows = lax.broadcasted_iota(jnp.int32, m.shape, 0)
        cols = lax.broadcasted_iota(jnp.int32, m.shape, 1)
        lin = rows * 128 + cols
        big = 2**30

        def body(t, mcur):
            gmax = jnp.max(mcur)
            sel = jnp.min(jnp.where(mcur >= gmax, lin, big))
            tidx_s[t] = sel
            return jnp.where(lin == sel, NEG, mcur)

        lax.fori_loop(0, TOPK, body, m)

        cps = [
            pltpu.make_async_copy(xg_ref.at[tidx_s[k]], topn_s.at[k], sem)
            for k in range(TOPK)
        ]
        for cp in cps:
            cp.start()
        for cp in cps:
            cp.wait()
        raw = topn_s[...]
        q = jnp.sum(raw * raw, axis=1, keepdims=True)
        topn_s[...] = raw / jnp.maximum(jnp.sqrt(q), EPS)

    @pl.when(jnp.logical_and(i > NBLK, i <= 2 * NBLK))
    def _matmul():
        j = i - NBLK - 1
        xb = x_ref[...]
        q = jnp.sum(xb * xb, axis=1, keepdims=True)
        xn = xb / jnp.maximum(jnp.sqrt(q), EPS)
        a = lax.dot_general(
            topn_s[...], xn, (((1,), (1,)), ((), ())),
            preferred_element_type=jnp.float32,
        )
        colid = lax.broadcasted_iota(jnp.int32, (1, ROWBLK), 1) + j * ROWBLK

        def body(t, excl):
            return jnp.where(colid == tidx_s[t], 1, excl)

        excl = lax.fori_loop(0, TOPK, body, jnp.zeros(colid.shape, jnp.int32))
        a_s[:, pl.ds(j * ROWBLK, ROWBLK)] = jnp.where(excl > 0, NEG, a)

    @pl.when(i == 2 * NBLK)
    def _top16():
        a = a_s[...]
        colidx = lax.broadcasted_iota(jnp.int32, a.shape, 1)
        big = 2**30

        def body(t, carry):
            acur, delmask = carry
            rowmax = jnp.max(acur, axis=1, keepdims=True)
            sel = jnp.min(
                jnp.where(acur >= rowmax, colidx, big), axis=1, keepdims=True
            )
            selmask = colidx == sel
            selany = jnp.max(
                jnp.where(selmask, 1.0, 0.0).astype(jnp.float32),
                axis=0, keepdims=True,
            )
            return jnp.where(selmask, NEG, acur), jnp.maximum(delmask, selany)

        _, delmask = lax.fori_loop(
            0, NSIM, body, (a, jnp.zeros((1, N), jnp.float32))
        )

        colid1 = lax.broadcasted_iota(jnp.int32, (1, N), 1)

        def tbody(t, dm):
            return jnp.where(colid1 == tidx_s[t], 1.0, dm)

        dropped = lax.fori_loop(0, TOPK, tbody, delmask)
        kept = jnp.sum(1.0 - dropped)
        ratio = N / kept
        scale_ref[...] = jnp.where(dropped > 0.0, 0.0, ratio)


def _scale_body(x_ref, s_ref, out_ref):
    out_ref[...] = x_ref[...] * s_ref[...]


def _x_index(i):
    return (
        jnp.where(i < NBLK, i, jnp.where(i == NBLK, NBLK - 1, i - NBLK - 1)),
        0,
    )


@jax.jit
def kernel(input):
    x = input

    scale = pl.pallas_call(
        _fused_body,
        grid=(2 * NBLK + 1,),
        in_specs=[
            pl.BlockSpec((ROWBLK, F), _x_index),
            pl.BlockSpec(memory_space=pl.ANY),
        ],
        out_specs=pl.BlockSpec((1, N), lambda i: (0, 0)),
        out_shape=jax.ShapeDtypeStruct((1, N), jnp.float32),
        scratch_shapes=[
            pltpu.VMEM((64, 128), jnp.float32),
            pltpu.VMEM((TOPK, F), jnp.float32),
            pltpu.VMEM((TOPK, N), jnp.float32),
            pltpu.SMEM((TOPK,), jnp.int32),
            pltpu.SemaphoreType.DMA,
        ],
    )(x, x)

    out = pl.pallas_call(
        _scale_body,
        grid=(NBLK,),
        in_specs=[
            pl.BlockSpec((ROWBLK, F), lambda i: (i, 0)),
            pl.BlockSpec((ROWBLK, 1), lambda i: (i, 0)),
        ],
        out_specs=pl.BlockSpec((ROWBLK, F), lambda i: (i, 0)),
        out_shape=jax.ShapeDtypeStruct((N, F), jnp.float32),
    )(x, scale.reshape(N, 1))

    return out


# speedup vs baseline: 1.0519x; 1.0519x over previous
import jax
import jax.numpy as jnp
from jax import lax
from jax.experimental import pallas as pl
from jax.experimental.pallas import tpu as pltpu

N = 8192
F = 2048
TOPK = 64
NSIM = 16
EPS = 1e-12
ROWBLK = 1024
NBLK = N // ROWBLK
NEG = float("-inf")


def _fused_body(x_ref, xg_ref, out_ref, means_s, topn_s, a_s, tidx_s, scale_s, sem):
    i = pl.program_id(0)

    @pl.when(i < NBLK)
    def _stats():
        xb = x_ref[...]
        s = jnp.sum(xb, axis=1, keepdims=True)
        m2 = jnp.reshape(s * (1.0 / F), (ROWBLK // 128, 128))
        means_s[pl.ds(i * (ROWBLK // 128), ROWBLK // 128), :] = m2

    @pl.when(i == NBLK)
    def _select_gather():
        m = means_s[...]
        rows = lax.broadcasted_iota(jnp.int32, m.shape, 0)
        cols = lax.broadcasted_iota(jnp.int32, m.shape, 1)
        lin = rows * 128 + cols
        big = 2**30

        def body(t, mcur):
            gmax = jnp.max(mcur)
            sel = jnp.min(jnp.where(mcur >= gmax, lin, big))
            tidx_s[t] = sel
            return jnp.where(lin == sel, NEG, mcur)

        lax.fori_loop(0, TOPK, body, m)

        cps = [
            pltpu.make_async_copy(xg_ref.at[tidx_s[k]], topn_s.at[k], sem)
            for k in range(TOPK)
        ]
        for cp in cps:
            cp.start()
        for cp in cps:
            cp.wait()
        raw = topn_s[...]
        q = jnp.sum(raw * raw, axis=1, keepdims=True)
        topn_s[...] = raw / jnp.maximum(jnp.sqrt(q), EPS)

    @pl.when(jnp.logical_and(i > NBLK, i <= 2 * NBLK))
    def _matmul():
        j = i - NBLK - 1
        xb = x_ref[...]
        q = jnp.sum(xb * xb, axis=1, keepdims=True)
        xn = xb / jnp.maximum(jnp.sqrt(q), EPS)
        a = lax.dot_general(
            topn_s[...], xn, (((1,), (1,)), ((), ())),
            preferred_element_type=jnp.float32,
        )
        colid = lax.broadcasted_iota(jnp.int32, (1, ROWBLK), 1) + j * ROWBLK

        def body(t, excl):
            return jnp.where(colid == tidx_s[t], 1, excl)

        excl = lax.fori_loop(0, TOPK, body, jnp.zeros(colid.shape, jnp.int32))
        a_s[:, pl.ds(j * ROWBLK, ROWBLK)] = jnp.where(excl > 0, NEG, a)

    @pl.when(i == 2 * NBLK)
    def _top16():
        a = a_s[...]
        colidx = lax.broadcasted_iota(jnp.int32, a.shape, 1)
        big = 2**30

        def body(t, carry):
            acur, delmask = carry
            rowmax = jnp.max(acur, axis=1, keepdims=True)
            sel = jnp.min(
                jnp.where(acur >= rowmax, colidx, big), axis=1, keepdims=True
            )
            selmask = colidx == sel
            selany = jnp.max(
                jnp.where(selmask, 1.0, 0.0).astype(jnp.float32),
                axis=0, keepdims=True,
            )
            return jnp.where(selmask, NEG, acur), jnp.maximum(delmask, selany)

        _, delmask = lax.fori_loop(
            0, NSIM, body, (a, jnp.zeros((1, N), jnp.float32))
        )

        colid1 = lax.broadcasted_iota(jnp.int32, (1, N), 1)

        def tbody(t, dm):
            return jnp.where(colid1 == tidx_s[t], 1.0, dm)

        dropped = lax.fori_loop(0, TOPK, tbody, delmask)
        kept = jnp.sum(1.0 - dropped)
        ratio = N / kept
        scale_s[...] = jnp.reshape(
            jnp.where(dropped > 0.0, 0.0, ratio), (N, 1)
        )

    @pl.when(i > 2 * NBLK)
    def _apply():
        j = i - 2 * NBLK - 1
        srow = scale_s[pl.ds(j * ROWBLK, ROWBLK), :]
        out_ref[...] = x_ref[...] * srow


def _x_index(i):
    blk = jnp.where(
        i < NBLK,
        i,
        jnp.where(
            i == NBLK,
            NBLK - 1,
            jnp.where(i <= 2 * NBLK, i - NBLK - 1, i - 2 * NBLK - 1),
        ),
    )
    return (blk, 0)


def _out_index(i):
    return (jnp.where(i <= 2 * NBLK, 0, i - 2 * NBLK - 1), 0)


@jax.jit
def kernel(input):
    x = input

    out = pl.pallas_call(
        _fused_body,
        grid=(3 * NBLK + 1,),
        in_specs=[
            pl.BlockSpec((ROWBLK, F), _x_index),
            pl.BlockSpec(memory_space=pl.ANY),
        ],
        out_specs=pl.BlockSpec((ROWBLK, F), _out_index),
        out_shape=jax.ShapeDtypeStruct((N, F), jnp.float32),
        scratch_shapes=[
            pltpu.VMEM((64, 128), jnp.float32),
            pltpu.VMEM((TOPK, F), jnp.float32),
            pltpu.VMEM((TOPK, N), jnp.float32),
            pltpu.SMEM((TOPK,), jnp.int32),
            pltpu.VMEM((N, 1), jnp.float32),
            pltpu.SemaphoreType.DMA,
        ],
    )(x, x)

    return out
